# initial kernel scaffold (unmeasured)
import jax
import jax.numpy as jnp
from jax import lax
from jax.experimental import pallas as pl
from jax.experimental.pallas import tpu as pltpu


def kernel(
    x,
):
    def body(*refs):
        pass

    out_shape = jax.ShapeDtypeStruct(..., jnp.float32)
    return pl.pallas_call(body, out_shape=out_shape)(...)


# baseline (device time: 176642 ns/iter reference)
import jax
import jax.numpy as jnp
from jax import lax
from jax.experimental import pallas as pl
from jax.experimental.pallas import tpu as pltpu

M, N = 2048, 1024
MESH = pl.DeviceIdType.MESH


def kernel(x):
    def body(x_ref, out_ref, scratch, send_sems, recv_sems):
        mx = lax.axis_index("x")
        my = lax.axis_index("y")
        mz = lax.axis_index("z")
        px = (1 - mx, my, mz)
        py = (mx, 1 - my, mz)
        pz = (mx, my, 1 - mz)

        barrier = pltpu.get_barrier_semaphore()
        for p in (px, py, pz):
            pl.semaphore_signal(barrier, inc=1, device_id=p, device_id_type=MESH)
        pl.semaphore_wait(barrier, 3)

        out_ref[:, :] = x_ref[:, :]

        kx = mx * 1024
        ky = kx + my * 512
        kz = ky + mz * 256

        rs_phases = [
            ((1 - mx) * 1024, kx, 1024, px, 0),
            (kx + (1 - my) * 512, ky, 512, py, 1024),
            (ky + (1 - mz) * 256, kz, 256, pz, 1536),
        ]
        for i, (s0, k0, sz, p, so) in enumerate(rs_phases):
            rdma = pltpu.make_async_remote_copy(
                src_ref=out_ref.at[pl.ds(s0, sz), :],
                dst_ref=scratch.at[pl.ds(so, sz), :],
                send_sem=send_sems.at[i],
                recv_sem=recv_sems.at[i],
                device_id=p,
                device_id_type=MESH,
            )
            rdma.start()
            rdma.wait()
            out_ref[pl.ds(k0, sz), :] = (
                out_ref[pl.ds(k0, sz), :] + scratch[pl.ds(so, sz), :]
            )

        ag_phases = [(kz, 256, pz), (ky, 512, py), (kx, 1024, px)]
        for j, (k0, sz, p) in enumerate(ag_phases):
            i = 3 + j
            rdma = pltpu.make_async_remote_copy(
                src_ref=out_ref.at[pl.ds(k0, sz), :],
                dst_ref=out_ref.at[pl.ds(k0, sz), :],
                send_sem=send_sems.at[i],
                recv_sem=recv_sems.at[i],
                device_id=p,
                device_id_type=MESH,
            )
            rdma.start()
            rdma.wait()

    out_shape = jax.ShapeDtypeStruct((M, N), jnp.float32)
    return pl.pallas_call(
        body,
        out_shape=out_shape,
        in_specs=[pl.BlockSpec(memory_space=pltpu.VMEM)],
        out_specs=pl.BlockSpec(memory_space=pltpu.VMEM),
        scratch_shapes=[
            pltpu.VMEM((1792, N), jnp.float32),
            pltpu.SemaphoreType.DMA((6,)),
            pltpu.SemaphoreType.DMA((6,)),
        ],
        compiler_params=pltpu.CompilerParams(collective_id=0),
    )(x.reshape(M, N))


# device time: 81411 ns/iter; 2.1698x vs baseline; 2.1698x over previous
import jax
import jax.numpy as jnp
from jax import lax
from jax.experimental import pallas as pl
from jax.experimental.pallas import tpu as pltpu

M, N = 2048, 1024
MESH = pl.DeviceIdType.MESH

SCHEDULES = [
    (0, 768, ("x", "y", "z")),
    (768, 768, ("y", "z", "x")),
    (1536, 512, ("z", "x", "y")),
]


def kernel(x):
    def body(x_ref, out_ref, scratch, send_sems, recv_sems):
        mx = lax.axis_index("x")
        my = lax.axis_index("y")
        mz = lax.axis_index("z")
        bits = {"x": mx, "y": my, "z": mz}
        partner = {
            "x": (1 - mx, my, mz),
            "y": (mx, 1 - my, mz),
            "z": (mx, my, 1 - mz),
        }

        barrier = pltpu.get_barrier_semaphore()
        for ax in ("x", "y", "z"):
            pl.semaphore_signal(
                barrier, inc=1, device_id=partner[ax], device_id_type=MESH
            )
        pl.semaphore_wait(barrier, 3)

        out_ref[:, :] = x_ref[:, :]

        scr_off = 0
        parts = []
        for off, rows, order in SCHEDULES:
            k = off
            ks = [k]
            rs = []
            for j, ax in enumerate(order):
                half = rows >> (j + 1)
                send0 = k + (1 - bits[ax]) * half
                k = k + bits[ax] * half
                ks.append(k)
                rs.append((ax, send0, k, half, scr_off))
                scr_off += half
            ag = [
                (order[2 - t], ks[3 - t], rows >> (3 - t)) for t in range(3)
            ]
            parts.append((rs, ag))

        for j in range(3):
            started = []
            for p_idx, (rs, _) in enumerate(parts):
                ax, send0, keep0, half, so = rs[j]
                rdma = pltpu.make_async_remote_copy(
                    src_ref=out_ref.at[pl.ds(send0, half), :],
                    dst_ref=scratch.at[pl.ds(so, half), :],
                    send_sem=send_sems.at[p_idx * 6 + j],
                    recv_sem=recv_sems.at[p_idx * 6 + j],
                    device_id=partner[ax],
                    device_id_type=MESH,
                )
                rdma.start()
                started.append((rdma, keep0, half, so))
            for rdma, keep0, half, so in started:
                rdma.wait()
                out_ref[pl.ds(keep0, half), :] = (
                    out_ref[pl.ds(keep0, half), :]
                    + scratch[pl.ds(so, half), :]
                )

        for t in range(3):
            started = []
            for p_idx, (_, ag) in enumerate(parts):
                ax, k0, sz = ag[t]
                rdma = pltpu.make_async_remote_copy(
                    src_ref=out_ref.at[pl.ds(k0, sz), :],
                    dst_ref=out_ref.at[pl.ds(k0, sz), :],
                    send_sem=send_sems.at[p_idx * 6 + 3 + t],
                    recv_sem=recv_sems.at[p_idx * 6 + 3 + t],
                    device_id=partner[ax],
                    device_id_type=MESH,
                )
                rdma.start()
                started.append(rdma)
            for rdma in started:
                rdma.wait()

    out_shape = jax.ShapeDtypeStruct((M, N), jnp.float32)
    return pl.pallas_call(
        body,
        out_shape=out_shape,
        in_specs=[pl.BlockSpec(memory_space=pltpu.VMEM)],
        out_specs=pl.BlockSpec(memory_space=pltpu.VMEM),
        scratch_shapes=[
            pltpu.VMEM((1792, N), jnp.float32),
            pltpu.SemaphoreType.DMA((18,)),
            pltpu.SemaphoreType.DMA((18,)),
        ],
        compiler_params=pltpu.CompilerParams(collective_id=0),
    )(x.reshape(M, N))


# device time: 75912 ns/iter; 2.3269x vs baseline; 1.0724x over previous
import jax
import jax.numpy as jnp
from jax import lax
from jax.experimental import pallas as pl
from jax.experimental.pallas import tpu as pltpu

M, N = 2048, 1024
MESH = pl.DeviceIdType.MESH

SCHEDULES = [
    (0, 704, ("x", "y", "z")),
    (704, 704, ("y", "z", "x")),
    (1408, 640, ("z", "x", "y")),
]


def kernel(x):
    def body(x_ref, out_ref, scratch, send_sems, recv_sems):
        mx = lax.axis_index("x")
        my = lax.axis_index("y")
        mz = lax.axis_index("z")
        bits = {"x": mx, "y": my, "z": mz}
        partner = {
            "x": (1 - mx, my, mz),
            "y": (mx, 1 - my, mz),
            "z": (mx, my, 1 - mz),
        }

        barrier = pltpu.get_barrier_semaphore()
        for ax in ("x", "y", "z"):
            pl.semaphore_signal(
                barrier, inc=1, device_id=partner[ax], device_id_type=MESH
            )
        pl.semaphore_wait(barrier, 3)


        scr_off = 0
        parts = []
        for off, rows, order in SCHEDULES:
            k = off
            ks = [k]
            rs = []
            for j, ax in enumerate(order):
                half = rows >> (j + 1)
                send0 = k + (1 - bits[ax]) * half
                k = k + bits[ax] * half
                ks.append(k)
                rs.append((ax, send0, k, half, scr_off))
                scr_off += half
            ag = [
                (order[2 - t], ks[3 - t], rows >> (3 - t)) for t in range(3)
            ]
            parts.append((rs, ag))

        for j in range(3):
            src = x_ref if j == 0 else out_ref
            started = []
            for p_idx, (rs, _) in enumerate(parts):
                ax, send0, keep0, half, so = rs[j]
                rdma = pltpu.make_async_remote_copy(
                    src_ref=src.at[pl.ds(send0, half), :],
                    dst_ref=scratch.at[pl.ds(so, half), :],
                    send_sem=send_sems.at[p_idx * 6 + j],
                    recv_sem=recv_sems.at[p_idx * 6 + j],
                    device_id=partner[ax],
                    device_id_type=MESH,
                )
                rdma.start()
                started.append((rdma, keep0, half, so))
            for rdma, keep0, half, so in started:
                rdma.wait()
                out_ref[pl.ds(keep0, half), :] = (
                    src[pl.ds(keep0, half), :]
                    + scratch[pl.ds(so, half), :]
                )

        for t in range(3):
            started = []
            for p_idx, (_, ag) in enumerate(parts):
                ax, k0, sz = ag[t]
                rdma = pltpu.make_async_remote_copy(
                    src_ref=out_ref.at[pl.ds(k0, sz), :],
                    dst_ref=out_ref.at[pl.ds(k0, sz), :],
                    send_sem=send_sems.at[p_idx * 6 + 3 + t],
                    recv_sem=recv_sems.at[p_idx * 6 + 3 + t],
                    device_id=partner[ax],
                    device_id_type=MESH,
                )
                rdma.start()
                started.append(rdma)
            for rdma in started:
                rdma.wait()

    out_shape = jax.ShapeDtypeStruct((M, N), jnp.float32)
    return pl.pallas_call(
        body,
        out_shape=out_shape,
        in_specs=[pl.BlockSpec(memory_space=pltpu.VMEM)],
        out_specs=pl.BlockSpec(memory_space=pltpu.VMEM),
        scratch_shapes=[
            pltpu.VMEM((1792, N), jnp.float32),
            pltpu.SemaphoreType.DMA((18,)),
            pltpu.SemaphoreType.DMA((18,)),
        ],
        compiler_params=pltpu.CompilerParams(collective_id=0),
    )(x.reshape(M, N))


# device time: 74332 ns/iter; 2.3764x vs baseline; 1.0213x over previous
import jax
import jax.numpy as jnp
from jax import lax
from jax.experimental import pallas as pl
from jax.experimental.pallas import tpu as pltpu

M, N = 2048, 1024
MESH = pl.DeviceIdType.MESH

SCHEDULES = [
    (0, 704, ("x", "y", "z")),
    (704, 704, ("y", "z", "x")),
    (1408, 640, ("z", "x", "y")),
]


def kernel(x):
    def body(x_ref, out_ref, scratch, send_sems, recv_sems):
        mx = lax.axis_index("x")
        my = lax.axis_index("y")
        mz = lax.axis_index("z")
        bits = {"x": mx, "y": my, "z": mz}
        partner = {
            "x": (1 - mx, my, mz),
            "y": (mx, 1 - my, mz),
            "z": (mx, my, 1 - mz),
        }

        barrier = pltpu.get_barrier_semaphore()
        for ax in ("x", "y", "z"):
            pl.semaphore_signal(
                barrier, inc=1, device_id=partner[ax], device_id_type=MESH
            )
        pl.semaphore_wait(barrier, 3)


        scr_off = 0
        parts = []
        for off, rows, order in SCHEDULES:
            k = off
            ks = [k]
            rs = []
            for j, ax in enumerate(order):
                half = rows >> (j + 1)
                send0 = k + (1 - bits[ax]) * half
                k = k + bits[ax] * half
                ks.append(k)
                rs.append((ax, send0, k, half, scr_off))
                scr_off += half
            ag = [
                (order[2 - t], ks[3 - t], rows >> (3 - t)) for t in range(3)
            ]
            parts.append((rs, ag))

        def start_rs(p_idx, j):
            rs = parts[p_idx][0]
            ax, send0, keep0, half, so = rs[j]
            src = x_ref if j == 0 else out_ref
            rdma = pltpu.make_async_remote_copy(
                src_ref=src.at[pl.ds(send0, half), :],
                dst_ref=scratch.at[pl.ds(so, half), :],
                send_sem=send_sems.at[p_idx * 6 + j],
                recv_sem=recv_sems.at[p_idx * 6 + j],
                device_id=partner[ax],
                device_id_type=MESH,
            )
            rdma.start()
            return rdma

        def start_ag(p_idx, t):
            ag = parts[p_idx][1]
            ax, k0, sz = ag[t]
            rdma = pltpu.make_async_remote_copy(
                src_ref=out_ref.at[pl.ds(k0, sz), :],
                dst_ref=out_ref.at[pl.ds(k0, sz), :],
                send_sem=send_sems.at[p_idx * 6 + 3 + t],
                recv_sem=recv_sems.at[p_idx * 6 + 3 + t],
                device_id=partner[ax],
                device_id_type=MESH,
            )
            rdma.start()
            return rdma

        inflight = []

        cur = [start_rs(p, 0) for p in range(3)]
        for j in range(3):
            src = x_ref if j == 0 else out_ref
            nxt = []
            for p_idx in range(3):
                rdma = cur[p_idx]
                rdma.wait_recv()
                inflight.append(rdma)
                _, _, keep0, half, so = parts[p_idx][0][j]
                out_ref[pl.ds(keep0, half), :] = (
                    src[pl.ds(keep0, half), :] + scratch[pl.ds(so, half), :]
                )
                nxt.append(
                    start_rs(p_idx, j + 1) if j < 2 else start_ag(p_idx, 0)
                )
            cur = nxt

        for t in range(3):
            nxt = []
            for p_idx in range(3):
                rdma = cur[p_idx]
                rdma.wait_recv()
                inflight.append(rdma)
                if t < 2:
                    nxt.append(start_ag(p_idx, t + 1))
            cur = nxt

        for rdma in inflight:
            rdma.wait_send()

    out_shape = jax.ShapeDtypeStruct((M, N), jnp.float32)
    return pl.pallas_call(
        body,
        out_shape=out_shape,
        in_specs=[pl.BlockSpec(memory_space=pltpu.VMEM)],
        out_specs=pl.BlockSpec(memory_space=pltpu.VMEM),
        scratch_shapes=[
            pltpu.VMEM((1792, N), jnp.float32),
            pltpu.SemaphoreType.DMA((18,)),
            pltpu.SemaphoreType.DMA((18,)),
        ],
        compiler_params=pltpu.CompilerParams(collective_id=0),
    )(x.reshape(M, N))


# device time: 71928 ns/iter; 2.4558x vs baseline; 1.0334x over previous
import jax
import jax.numpy as jnp
from jax import lax
from jax.experimental import pallas as pl
from jax.experimental.pallas import tpu as pltpu

M, N = 2048, 1024
MESH = pl.DeviceIdType.MESH

SCHEDULES = [
    (0, 704, ("x", "y", "z")),
    (704, 704, ("y", "z", "x")),
    (1408, 640, ("z", "x", "y")),
]

SEMS_PER_PART = 10


def kernel(x):
    def body(x_ref, out_ref, scratch, send_sems, recv_sems):
        mx = lax.axis_index("x")
        my = lax.axis_index("y")
        mz = lax.axis_index("z")
        bits = {"x": mx, "y": my, "z": mz}
        partner = {
            "x": (1 - mx, my, mz),
            "y": (mx, 1 - my, mz),
            "z": (mx, my, 1 - mz),
        }

        barrier = pltpu.get_barrier_semaphore()
        for ax in ("x", "y", "z"):
            pl.semaphore_signal(
                barrier, inc=1, device_id=partner[ax], device_id_type=MESH
            )
        pl.semaphore_wait(barrier, 3)

        def blk_off(p_idx, flips=()):
            off, rows, order = SCHEDULES[p_idx]
            k = off
            for j, ax in enumerate(order):
                b = (1 - bits[ax]) if ax in flips else bits[ax]
                k = k + b * (rows >> (j + 1))
            return k

        scr_off = 0
        parts = []
        for off, rows, order in SCHEDULES:
            k = off
            rs = []
            for j, ax in enumerate(order):
                half = rows >> (j + 1)
                send0 = k + (1 - bits[ax]) * half
                k = k + bits[ax] * half
                rs.append((ax, send0, k, half, scr_off))
                scr_off += half
            parts.append(rs)

        inflight = []

        def start_rs(p_idx, j):
            ax, send0, _, half, so = parts[p_idx][j]
            src = x_ref if j == 0 else out_ref
            rdma = pltpu.make_async_remote_copy(
                src_ref=src.at[pl.ds(send0, half), :],
                dst_ref=scratch.at[pl.ds(so, half), :],
                send_sem=send_sems.at[p_idx * SEMS_PER_PART + j],
                recv_sem=recv_sems.at[p_idx * SEMS_PER_PART + j],
                device_id=partner[ax],
                device_id_type=MESH,
            )
            rdma.start()
            return rdma

        def send_blk(p_idx, flips, ax, slot):
            e = SCHEDULES[p_idx][1] >> 3
            k0 = blk_off(p_idx, flips)
            rdma = pltpu.make_async_remote_copy(
                src_ref=out_ref.at[pl.ds(k0, e), :],
                dst_ref=out_ref.at[pl.ds(k0, e), :],
                send_sem=send_sems.at[p_idx * SEMS_PER_PART + slot],
                recv_sem=recv_sems.at[p_idx * SEMS_PER_PART + slot],
                device_id=partner[ax],
                device_id_type=MESH,
            )
            rdma.start()
            inflight.append(rdma)
            return rdma

        def bcast_axes(p_idx):
            order = SCHEDULES[p_idx][2]
            return order[2], order[1], order[0]

        cur = [start_rs(p, 0) for p in range(3)]
        ag0 = []
        for j in range(3):
            src = x_ref if j == 0 else out_ref
            nxt = []
            for p_idx in range(3):
                rdma = cur[p_idx]
                rdma.wait_recv()
                inflight.append(rdma)
                _, _, keep0, half, so = parts[p_idx][j]
                out_ref[pl.ds(keep0, half), :] = (
                    src[pl.ds(keep0, half), :] + scratch[pl.ds(so, half), :]
                )
                if j < 2:
                    nxt.append(start_rs(p_idx, j + 1))
                else:
                    s0, s1, s2 = bcast_axes(p_idx)
                    r3 = send_blk(p_idx, (), s0, 3)
                    send_blk(p_idx, (), s1, 4)
                    send_blk(p_idx, (), s2, 6)
                    ag0.append(r3)
            cur = nxt

        fw_s2 = []
        for p_idx in range(3):
            s0, s1, s2 = bcast_axes(p_idx)
            ag0[p_idx].wait_recv()
            send_blk(p_idx, (s0,), s1, 5)
            send_blk(p_idx, (s0,), s2, 7)
        s1_recvs = []
        for p_idx in range(3):
            s0, s1, s2 = bcast_axes(p_idx)
            e = SCHEDULES[p_idx][1] >> 3
            for slot, flips, fwd_slot in (
                (4, (s1,), 8),
                (5, (s1, s0), 9),
            ):
                r = pltpu.make_async_remote_copy(
                    src_ref=out_ref.at[pl.ds(blk_off(p_idx, flips), e), :],
                    dst_ref=out_ref.at[pl.ds(blk_off(p_idx, flips), e), :],
                    send_sem=send_sems.at[p_idx * SEMS_PER_PART + slot],
                    recv_sem=recv_sems.at[p_idx * SEMS_PER_PART + slot],
                    device_id=partner[s1],
                    device_id_type=MESH,
                )
                r.wait_recv()
                send_blk(p_idx, flips, s2, fwd_slot)
        for p_idx in range(3):
            s0, s1, s2 = bcast_axes(p_idx)
            e = SCHEDULES[p_idx][1] >> 3
            for slot, flips in (
                (6, (s2,)),
                (7, (s2, s0)),
                (8, (s2, s1)),
                (9, (s2, s1, s0)),
            ):
                r = pltpu.make_async_remote_copy(
                    src_ref=out_ref.at[pl.ds(blk_off(p_idx, flips), e), :],
                    dst_ref=out_ref.at[pl.ds(blk_off(p_idx, flips), e), :],
                    send_sem=send_sems.at[p_idx * SEMS_PER_PART + slot],
                    recv_sem=recv_sems.at[p_idx * SEMS_PER_PART + slot],
                    device_id=partner[s2],
                    device_id_type=MESH,
                )
                r.wait_recv()

        for rdma in inflight:
            rdma.wait_send()

    out_shape = jax.ShapeDtypeStruct((M, N), jnp.float32)
    return pl.pallas_call(
        body,
        out_shape=out_shape,
        in_specs=[pl.BlockSpec(memory_space=pltpu.VMEM)],
        out_specs=pl.BlockSpec(memory_space=pltpu.VMEM),
        scratch_shapes=[
            pltpu.VMEM((1792, N), jnp.float32),
            pltpu.SemaphoreType.DMA((3 * SEMS_PER_PART,)),
            pltpu.SemaphoreType.DMA((3 * SEMS_PER_PART,)),
        ],
        compiler_params=pltpu.CompilerParams(collective_id=0),
    )(x.reshape(M, N))


# device time: 67196 ns/iter; 2.6288x vs baseline; 1.0704x over previous
import jax
import jax.numpy as jnp
from jax import lax
from jax.experimental import pallas as pl
from jax.experimental.pallas import tpu as pltpu

M, N = 2048, 1024
MESH = pl.DeviceIdType.MESH

SCHEDULES = [
    (0, 704, ("x", "y", "z")),
    (704, 704, ("y", "z", "x")),
    (1408, 640, ("z", "x", "y")),
]

SEMS_PER_PART = 13


def kernel(x):
    def body(x_ref, out_ref, scratch, send_sems, recv_sems):
        mx = lax.axis_index("x")
        my = lax.axis_index("y")
        mz = lax.axis_index("z")
        bits = {"x": mx, "y": my, "z": mz}
        partner = {
            "x": (1 - mx, my, mz),
            "y": (mx, 1 - my, mz),
            "z": (mx, my, 1 - mz),
        }

        barrier = pltpu.get_barrier_semaphore()
        for ax in ("x", "y", "z"):
            pl.semaphore_signal(
                barrier, inc=1, device_id=partner[ax], device_id_type=MESH
            )
        pl.semaphore_wait(barrier, 3)

        def blk_off(p_idx, flips=()):
            off, rows, order = SCHEDULES[p_idx]
            k = off
            for j, ax in enumerate(order):
                b = (1 - bits[ax]) if ax in flips else bits[ax]
                k = k + b * (rows >> (j + 1))
            return k

        geoms = []
        scr_base = 0
        for off, rows, order in SCHEDULES:
            b0, b1, b2 = (bits[a] for a in order)
            h, q, e = rows >> 1, rows >> 2, rows >> 3
            k1 = off + b0 * h
            k2 = k1 + b1 * q
            k3 = k2 + b2 * e
            send0 = off + (1 - b0) * h
            s1 = k1 + (1 - b1) * q
            s2 = k2 + (1 - b2) * e
            r_s1 = (1 - b1) * q
            r_s2 = b1 * q + (1 - b2) * e
            r_k = b1 * q + b2 * e
            r1_s2 = (1 - b2) * e
            r1_k = b2 * e
            geoms.append(
                dict(
                    order=order, h=h, q=q, e=e,
                    k1=k1, k2=k2, k3=k3, send0=send0, s1=s1, s2=s2,
                    r_s1=r_s1, r_s2=r_s2, r_k=r_k, r1_s2=r1_s2, r1_k=r1_k,
                    scr0=scr_base, scr1=scr_base + h, scr2=scr_base + h + q,
                )
            )
            scr_base += h + q + e

        inflight = []

        def copy(p_idx, slot, ax, src_ref, s0, dst_ref, d0, sz):
            rdma = pltpu.make_async_remote_copy(
                src_ref=src_ref.at[pl.ds(s0, sz), :],
                dst_ref=dst_ref.at[pl.ds(d0, sz), :],
                send_sem=send_sems.at[p_idx * SEMS_PER_PART + slot],
                recv_sem=recv_sems.at[p_idx * SEMS_PER_PART + slot],
                device_id=partner[ax],
                device_id_type=MESH,
            )
            rdma.start()
            inflight.append(rdma)
            return rdma

        def send_blk(p_idx, flips, ax, slot):
            e = geoms[p_idx]["e"]
            k0 = blk_off(p_idx, flips)
            return copy(p_idx, slot, ax, out_ref, k0, out_ref, k0, e)

        def bcast_axes(p_idx):
            order = SCHEDULES[p_idx][2]
            return order[2], order[1], order[0]

        r0 = []
        for p_idx, g in enumerate(geoms):
            a0 = g["order"][0]
            c1 = copy(p_idx, 0, a0, x_ref, g["send0"] + g["r_s1"],
                      scratch, g["scr0"] + g["r_s1"], g["q"])
            c2 = copy(p_idx, 1, a0, x_ref, g["send0"] + g["r_s2"],
                      scratch, g["scr0"] + g["r_s2"], g["e"])
            c3 = copy(p_idx, 2, a0, x_ref, g["send0"] + g["r_k"],
                      scratch, g["scr0"] + g["r_k"], g["e"])
            r0.append((c1, c2, c3))

        r1 = []
        for p_idx, g in enumerate(geoms):
            a1 = g["order"][1]
            r0[p_idx][0].wait_recv()
            out_ref[pl.ds(g["s1"], g["q"]), :] = (
                x_ref[pl.ds(g["s1"], g["q"]), :]
                + scratch[pl.ds(g["scr0"] + g["r_s1"], g["q"]), :]
            )
            c1 = copy(p_idx, 3, a1, out_ref, g["s1"] + g["r1_s2"],
                      scratch, g["scr1"] + g["r1_s2"], g["e"])
            c2 = copy(p_idx, 4, a1, out_ref, g["s1"] + g["r1_k"],
                      scratch, g["scr1"] + g["r1_k"], g["e"])
            r1.append((c1, c2))

        r2 = []
        for p_idx, g in enumerate(geoms):
            a2 = g["order"][2]
            r1[p_idx][0].wait_recv()
            r0[p_idx][1].wait_recv()
            out_ref[pl.ds(g["s2"], g["e"]), :] = (
                x_ref[pl.ds(g["s2"], g["e"]), :]
                + scratch[pl.ds(g["scr0"] + g["r_s2"], g["e"]), :]
                + scratch[pl.ds(g["scr1"] + g["r1_s2"], g["e"]), :]
            )
            r2.append(
                copy(p_idx, 5, a2, out_ref, g["s2"], scratch, g["scr2"],
                     g["e"])
            )

        ag0 = []
        for p_idx, g in enumerate(geoms):
            r2[p_idx].wait_recv()
            r0[p_idx][2].wait_recv()
            r1[p_idx][1].wait_recv()
            out_ref[pl.ds(g["k3"], g["e"]), :] = (
                x_ref[pl.ds(g["k3"], g["e"]), :]
                + scratch[pl.ds(g["scr0"] + g["r_k"], g["e"]), :]
                + scratch[pl.ds(g["scr1"] + g["r1_k"], g["e"]), :]
                + scratch[pl.ds(g["scr2"], g["e"]), :]
            )
            s0, s1, s2 = bcast_axes(p_idx)
            ag0.append(send_blk(p_idx, (), s0, 6))
            send_blk(p_idx, (), s1, 7)
            send_blk(p_idx, (), s2, 9)

        for p_idx in range(3):
            s0, s1, s2 = bcast_axes(p_idx)
            ag0[p_idx].wait_recv()
            send_blk(p_idx, (s0,), s1, 8)
            send_blk(p_idx, (s0,), s2, 10)
        for p_idx, g in enumerate(geoms):
            s0, s1, s2 = bcast_axes(p_idx)
            for slot, flips, fwd_slot in ((7, (s1,), 11), (8, (s1, s0), 12)):
                k0 = blk_off(p_idx, flips)
                r = pltpu.make_async_remote_copy(
                    src_ref=out_ref.at[pl.ds(k0, g["e"]), :],
                    dst_ref=out_ref.at[pl.ds(k0, g["e"]), :],
                    send_sem=send_sems.at[p_idx * SEMS_PER_PART + slot],
                    recv_sem=recv_sems.at[p_idx * SEMS_PER_PART + slot],
                    device_id=partner[s1],
                    device_id_type=MESH,
                )
                r.wait_recv()
                send_blk(p_idx, flips, s2, fwd_slot)
        for p_idx, g in enumerate(geoms):
            s0, s1, s2 = bcast_axes(p_idx)
            for slot, flips in (
                (9, (s2,)),
                (10, (s2, s0)),
                (11, (s2, s1)),
                (12, (s2, s1, s0)),
            ):
                k0 = blk_off(p_idx, flips)
                r = pltpu.make_async_remote_copy(
                    src_ref=out_ref.at[pl.ds(k0, g["e"]), :],
                    dst_ref=out_ref.at[pl.ds(k0, g["e"]), :],
                    send_sem=send_sems.at[p_idx * SEMS_PER_PART + slot],
                    recv_sem=recv_sems.at[p_idx * SEMS_PER_PART + slot],
                    device_id=partner[s2],
                    device_id_type=MESH,
                )
                r.wait_recv()

        for rdma in inflight:
            rdma.wait_send()

    out_shape = jax.ShapeDtypeStruct((M, N), jnp.float32)
    return pl.pallas_call(
        body,
        out_shape=out_shape,
        in_specs=[pl.BlockSpec(memory_space=pltpu.VMEM)],
        out_specs=pl.BlockSpec(memory_space=pltpu.VMEM),
        scratch_shapes=[
            pltpu.VMEM((1792, N), jnp.float32),
            pltpu.SemaphoreType.DMA((3 * SEMS_PER_PART,)),
            pltpu.SemaphoreType.DMA((3 * SEMS_PER_PART,)),
        ],
        compiler_params=pltpu.CompilerParams(collective_id=0),
    )(x.reshape(M, N))
